# Initial kernel scaffold; baseline (speedup 1.0000x reference)
import jax
import jax.numpy as jnp
from jax.experimental import pallas as pl


def kernel(x, edge_index, batch, batch_size, Wl1, Wr1, b1, Wl2, Wr2, b2):
    raise NotImplementedError("write your pallas kernel here")


# speedup vs baseline: 7.3551x; 7.3551x over previous
import functools

import jax
import jax.numpy as jnp
from jax import lax
from jax.experimental import pallas as pl
from jax.experimental.pallas import tpu as pltpu
from jax.experimental.pallas import tpu_sc as plsc

N = 10000
E = 320000
D = 128
B = 64
DC = 16

NC = 2
NS = 16
NT = NC * NS
EPT = E // NT
CH = 80
NCHUNK = EPT // CH
RPT = N // NS
ZR = 25

RB = 1000
G1 = N // RB

_HIGH = lax.Precision.HIGHEST


@functools.lru_cache(maxsize=None)
def _make_sc_agg(width):
    mesh = plsc.VectorSubcoreMesh(core_axis_name="c", subcore_axis_name="s")

    @functools.partial(
        pl.kernel,
        out_type=jax.ShapeDtypeStruct((NC, N, width), jnp.float32),
        mesh=mesh,
        compiler_params=pltpu.CompilerParams(use_tc_tiling_on_sc=False),
        scratch_types=[
            pltpu.VMEM((NCHUNK, CH), jnp.int32),
            pltpu.VMEM((NCHUNK, CH), jnp.int32),
            pltpu.VMEM((CH, width), jnp.float32),
            pltpu.VMEM((ZR, width), jnp.float32),
            pltpu.VMEM_SHARED((N, width), jnp.float32),
            pltpu.SemaphoreType.DMA,
        ],
    )
    def agg(xa_hbm, src_hbm, dst_hbm, out_hbm, srcv, dstv, rows, zbuf, acc, sem):
        c = lax.axis_index("c")
        s = lax.axis_index("s")
        tid = c * NS + s

        pltpu.sync_copy(src_hbm.at[tid], srcv)
        pltpu.sync_copy(dst_hbm.at[tid], dstv)

        nv = width // 16

        def zrow(i, carry):
            for j in range(nv):
                zbuf[i, pl.ds(j * 16, 16)] = jnp.zeros((16,), jnp.float32)
            return carry

        lax.fori_loop(0, ZR, zrow, 0)
        for r in range(RPT // ZR):
            pltpu.sync_copy(zbuf, acc.at[pl.ds(s * RPT + r * ZR, ZR)])
        plsc.subcore_barrier()

        def body(i, carry):
            pltpu.async_copy(xa_hbm.at[srcv.at[i]], rows, sem).wait()
            pltpu.sync_copy(rows, acc.at[dstv.at[i]], add=True)
            return carry

        lax.fori_loop(0, NCHUNK, body, 0)
        plsc.subcore_barrier()

        pltpu.sync_copy(acc.at[pl.ds(s * RPT, RPT)],
                        out_hbm.at[c, pl.ds(s * RPT, RPT)])

    return agg


def _tc_layer1(parts, x, WlT, WrT, b1):
    def body(p_ref, x_ref, wl_ref, wr_ref, b_ref, h_ref, rinv_ref):
        agg = p_ref[0, :, :D] + p_ref[1, :, :D]
        cnt = p_ref[0, :, D:D + 1] + p_ref[1, :, D:D + 1]
        rinv = 1.0 / jnp.maximum(cnt, 1.0)
        mean = agg * rinv
        h = jnp.dot(mean, wl_ref[...], precision=_HIGH,
                    preferred_element_type=jnp.float32)
        h += jnp.dot(x_ref[...], wr_ref[...], precision=_HIGH,
                     preferred_element_type=jnp.float32)
        h += b_ref[...]
        h_ref[...] = jnp.maximum(h, 0.0)
        rinv_ref[...] = jnp.broadcast_to(rinv, (RB, D))

    return pl.pallas_call(
        body,
        grid=(G1,),
        in_specs=[
            pl.BlockSpec((NC, RB, D + DC), lambda i: (0, i, 0)),
            pl.BlockSpec((RB, D), lambda i: (i, 0)),
            pl.BlockSpec((D, D), lambda i: (0, 0)),
            pl.BlockSpec((D, D), lambda i: (0, 0)),
            pl.BlockSpec((1, D), lambda i: (0, 0)),
        ],
        out_specs=[
            pl.BlockSpec((RB, D), lambda i: (i, 0)),
            pl.BlockSpec((RB, D), lambda i: (i, 0)),
        ],
        out_shape=[
            jax.ShapeDtypeStruct((N, D), jnp.float32),
            jax.ShapeDtypeStruct((N, D), jnp.float32),
        ],
    )(parts, x, WlT, WrT, b1)


def _tc_layer2(parts2, rinv, h, seg3, WlT, WrT, b2):
    def body(p_ref, rinv_ref, h_ref, seg_ref, wl_ref, wr_ref, b_ref,
             out_ref, pm, ph, po):
        i = pl.program_id(0)

        @pl.when(i == 0)
        def _():
            pm[...] = jnp.zeros((B, D), jnp.float32)
            ph[...] = jnp.zeros((B, D), jnp.float32)
            po[...] = jnp.zeros((B, D), jnp.float32)

        mean2 = (p_ref[0] + p_ref[1]) * rinv_ref[...]
        seg = seg_ref[0]
        ohT = (lax.broadcasted_iota(jnp.int32, (B, RB), 0) == seg
               ).astype(jnp.float32)
        pm[...] += jnp.dot(ohT, mean2, precision=_HIGH,
                           preferred_element_type=jnp.float32)
        ph[...] += jnp.dot(ohT, h_ref[...], precision=_HIGH,
                           preferred_element_type=jnp.float32)
        po[...] += jnp.dot(ohT, jnp.ones((RB, D), jnp.float32),
                           precision=_HIGH, preferred_element_type=jnp.float32)

        @pl.when(i == G1 - 1)
        def _():
            out_ref[...] = (
                jnp.dot(pm[...], wl_ref[...], precision=_HIGH,
                        preferred_element_type=jnp.float32)
                + jnp.dot(ph[...], wr_ref[...], precision=_HIGH,
                          preferred_element_type=jnp.float32)
                + po[...] * b_ref[...])

    return pl.pallas_call(
        body,
        grid=(G1,),
        in_specs=[
            pl.BlockSpec((NC, RB, D), lambda i: (0, i, 0)),
            pl.BlockSpec((RB, D), lambda i: (i, 0)),
            pl.BlockSpec((RB, D), lambda i: (i, 0)),
            pl.BlockSpec((1, 1, RB), lambda i: (i, 0, 0)),
            pl.BlockSpec((D, D), lambda i: (0, 0)),
            pl.BlockSpec((D, D), lambda i: (0, 0)),
            pl.BlockSpec((1, D), lambda i: (0, 0)),
        ],
        out_specs=pl.BlockSpec((B, D), lambda i: (0, 0)),
        out_shape=jax.ShapeDtypeStruct((B, D), jnp.float32),
        scratch_shapes=[
            pltpu.VMEM((B, D), jnp.float32),
            pltpu.VMEM((B, D), jnp.float32),
            pltpu.VMEM((B, D), jnp.float32),
        ],
    )(parts2, rinv, h, seg3, WlT, WrT, b2)


def kernel(x, edge_index, batch, batch_size, Wl1, Wr1, b1, Wl2, Wr2, b2):
    x = x.astype(jnp.float32)
    src = edge_index[0].astype(jnp.int32).reshape(NT, NCHUNK, CH)
    dst = edge_index[1].astype(jnp.int32).reshape(NT, NCHUNK, CH)
    seg3 = jnp.minimum(batch, batch_size - 1).astype(jnp.int32).reshape(G1, 1, RB)

    xa = jnp.concatenate([x, jnp.ones((N, DC), jnp.float32)], axis=1)
    parts1 = _make_sc_agg(D + DC)(xa, src, dst)
    h, rinv = _tc_layer1(parts1, x, Wl1.T, Wr1.T, b1.reshape(1, D))

    parts2 = _make_sc_agg(D)(h, src, dst)
    return _tc_layer2(parts2, rinv, h, seg3, Wl2.T, Wr2.T, b2.reshape(1, D))


# speedup vs baseline: 8.5954x; 1.1686x over previous
import functools

import jax
import jax.numpy as jnp
from jax import lax
from jax.experimental import pallas as pl
from jax.experimental.pallas import tpu as pltpu
from jax.experimental.pallas import tpu_sc as plsc

N = 10000
E = 320000
D = 128
B = 64
DC = 16

NC = 2
NS = 16
NT = NC * NS
EPT = E // NT
CH = 80
NCHUNK = EPT // CH
RPT = N // NS
ZR = 25

RB = 1000
G1 = N // RB

_HIGH = lax.Precision.HIGHEST


@functools.lru_cache(maxsize=None)
def _make_sc_agg(width):
    mesh = plsc.VectorSubcoreMesh(core_axis_name="c", subcore_axis_name="s")

    @functools.partial(
        pl.kernel,
        out_type=jax.ShapeDtypeStruct((NC, N, width), jnp.float32),
        mesh=mesh,
        compiler_params=pltpu.CompilerParams(use_tc_tiling_on_sc=False),
        scratch_types=[
            pltpu.VMEM((2, 2, CH), jnp.int32),
            pltpu.VMEM((2, CH, width), jnp.float32),
            pltpu.VMEM((ZR, width), jnp.float32),
            pltpu.VMEM_SHARED((N, width), jnp.float32),
            pltpu.SemaphoreType.DMA,
            pltpu.SemaphoreType.DMA,
            pltpu.SemaphoreType.DMA,
            pltpu.SemaphoreType.DMA,
        ],
    )
    def agg(xa_hbm, ei_hbm, out_hbm, idxr, rowsr, zbuf, acc,
            isem0, isem1, gsem0, gsem1):
        c = lax.axis_index("c")
        s = lax.axis_index("s")
        tid = c * NS + s
        isems = (isem0, isem1)
        gsems = (gsem0, gsem1)

        def idx_start(ch, slot):
            pltpu.async_copy(ei_hbm.at[tid, ch], idxr.at[slot], isems[slot])

        def idx_wait(ch, slot):
            pltpu.make_async_copy(ei_hbm.at[tid, ch], idxr.at[slot],
                                  isems[slot]).wait()

        def g_start(slot):
            pltpu.async_copy(xa_hbm.at[idxr.at[slot, 0]], rowsr.at[slot],
                             gsems[slot])

        def g_wait(slot):
            pltpu.make_async_copy(xa_hbm.at[idxr.at[slot, 0]], rowsr.at[slot],
                                  gsems[slot]).wait()

        def scat(slot):
            pltpu.sync_copy(rowsr.at[slot], acc.at[idxr.at[slot, 1]], add=True)

        idx_start(0, 0)
        idx_start(1, 1)
        idx_wait(0, 0)
        g_start(0)

        nv = width // 16

        def zrow(i, carry):
            for j in range(nv):
                zbuf[i, pl.ds(j * 16, 16)] = jnp.zeros((16,), jnp.float32)
            return carry

        lax.fori_loop(0, ZR, zrow, 0)
        for r in range(RPT // ZR):
            pltpu.sync_copy(zbuf, acc.at[pl.ds(s * RPT + r * ZR, ZR)])
        plsc.subcore_barrier()

        def body(i, carry):
            c0 = 2 * i
            c1 = c0 + 1
            g_wait(0)
            idx_wait(c1, 1)
            g_start(1)
            scat(0)
            idx_start(c0 + 2, 0)
            g_wait(1)
            idx_wait(c0 + 2, 0)
            g_start(0)
            scat(1)
            idx_start(c1 + 2, 1)
            return carry

        nbody = (NCHUNK - 5) // 2
        lax.fori_loop(0, nbody, body, 0)

        for ch in range(2 * nbody, NCHUNK):
            slot = ch % 2
            g_wait(slot)
            if ch + 1 < NCHUNK:
                idx_wait(ch + 1, (ch + 1) % 2)
                g_start((ch + 1) % 2)
            scat(slot)
            if ch + 2 < NCHUNK:
                idx_start(ch + 2, slot)
        plsc.subcore_barrier()

        pltpu.sync_copy(acc.at[pl.ds(s * RPT, RPT)],
                        out_hbm.at[c, pl.ds(s * RPT, RPT)])

    return agg


def _tc_layer1(parts, x, WlT, WrT, b1):
    def body(p_ref, x_ref, wl_ref, wr_ref, b_ref, h_ref, rinv_ref):
        agg = p_ref[0, :, :D] + p_ref[1, :, :D]
        cnt = p_ref[0, :, D:D + 1] + p_ref[1, :, D:D + 1]
        rinv = 1.0 / jnp.maximum(cnt, 1.0)
        mean = agg * rinv
        h = jnp.dot(mean, wl_ref[...], precision=_HIGH,
                    preferred_element_type=jnp.float32)
        h += jnp.dot(x_ref[...], wr_ref[...], precision=_HIGH,
                     preferred_element_type=jnp.float32)
        h += b_ref[...]
        h_ref[...] = jnp.maximum(h, 0.0)
        rinv_ref[...] = jnp.broadcast_to(rinv, (RB, D))

    return pl.pallas_call(
        body,
        grid=(G1,),
        in_specs=[
            pl.BlockSpec((NC, RB, D + DC), lambda i: (0, i, 0)),
            pl.BlockSpec((RB, D), lambda i: (i, 0)),
            pl.BlockSpec((D, D), lambda i: (0, 0)),
            pl.BlockSpec((D, D), lambda i: (0, 0)),
            pl.BlockSpec((1, D), lambda i: (0, 0)),
        ],
        out_specs=[
            pl.BlockSpec((RB, D), lambda i: (i, 0)),
            pl.BlockSpec((RB, D), lambda i: (i, 0)),
        ],
        out_shape=[
            jax.ShapeDtypeStruct((N, D), jnp.float32),
            jax.ShapeDtypeStruct((N, D), jnp.float32),
        ],
    )(parts, x, WlT, WrT, b1)


def _tc_layer2(parts2, rinv, h, seg3, WlT, WrT, b2):
    def body(p_ref, rinv_ref, h_ref, seg_ref, wl_ref, wr_ref, b_ref,
             out_ref, pm, ph, po):
        i = pl.program_id(0)

        @pl.when(i == 0)
        def _():
            pm[...] = jnp.zeros((B, D), jnp.float32)
            ph[...] = jnp.zeros((B, D), jnp.float32)
            po[...] = jnp.zeros((B, D), jnp.float32)

        mean2 = (p_ref[0] + p_ref[1]) * rinv_ref[...]
        seg = seg_ref[0]
        ohT = (lax.broadcasted_iota(jnp.int32, (B, RB), 0) == seg
               ).astype(jnp.float32)
        pm[...] += jnp.dot(ohT, mean2, precision=_HIGH,
                           preferred_element_type=jnp.float32)
        ph[...] += jnp.dot(ohT, h_ref[...], precision=_HIGH,
                           preferred_element_type=jnp.float32)
        po[...] += jnp.dot(ohT, jnp.ones((RB, D), jnp.float32),
                           precision=_HIGH, preferred_element_type=jnp.float32)

        @pl.when(i == G1 - 1)
        def _():
            out_ref[...] = (
                jnp.dot(pm[...], wl_ref[...], precision=_HIGH,
                        preferred_element_type=jnp.float32)
                + jnp.dot(ph[...], wr_ref[...], precision=_HIGH,
                          preferred_element_type=jnp.float32)
                + po[...] * b_ref[...])

    return pl.pallas_call(
        body,
        grid=(G1,),
        in_specs=[
            pl.BlockSpec((NC, RB, D), lambda i: (0, i, 0)),
            pl.BlockSpec((RB, D), lambda i: (i, 0)),
            pl.BlockSpec((RB, D), lambda i: (i, 0)),
            pl.BlockSpec((1, 1, RB), lambda i: (i, 0, 0)),
            pl.BlockSpec((D, D), lambda i: (0, 0)),
            pl.BlockSpec((D, D), lambda i: (0, 0)),
            pl.BlockSpec((1, D), lambda i: (0, 0)),
        ],
        out_specs=pl.BlockSpec((B, D), lambda i: (0, 0)),
        out_shape=jax.ShapeDtypeStruct((B, D), jnp.float32),
        scratch_shapes=[
            pltpu.VMEM((B, D), jnp.float32),
            pltpu.VMEM((B, D), jnp.float32),
            pltpu.VMEM((B, D), jnp.float32),
        ],
    )(parts2, rinv, h, seg3, WlT, WrT, b2)


def kernel(x, edge_index, batch, batch_size, Wl1, Wr1, b1, Wl2, Wr2, b2):
    x = x.astype(jnp.float32)
    ei = jnp.stack(
        [edge_index[0].astype(jnp.int32).reshape(NT, NCHUNK, CH),
         edge_index[1].astype(jnp.int32).reshape(NT, NCHUNK, CH)], axis=2)
    seg3 = jnp.minimum(batch, batch_size - 1).astype(jnp.int32).reshape(G1, 1, RB)

    xa = jnp.concatenate([x, jnp.ones((N, DC), jnp.float32)], axis=1)
    parts1 = _make_sc_agg(D + DC)(xa, ei)
    h, rinv = _tc_layer1(parts1, x, Wl1.T, Wr1.T, b1.reshape(1, D))

    parts2 = _make_sc_agg(D)(h, ei)
    return _tc_layer2(parts2, rinv, h, seg3, Wl2.T, Wr2.T, b2.reshape(1, D))
